# baseline (speedup 1.0000x reference)
import functools

import jax
import jax.numpy as jnp
from jax import lax
from jax.experimental import pallas as pl
from jax.experimental.pallas import tpu as pltpu
from jax.experimental.pallas import tpu_sc as plsc

_B, _T, _K = 128, 2048, 5
_C = 8
_R = _C * _K
_NHALF = _T // (2 * _C)
_NW = 32
_BPW = _B // _NW


def _post_sc_body(em_hbm, tags_hbm, tbl_hbm, out_hbm, em_v, tg_v, tbl_v,
                  out_v):
    wid = lax.axis_index("s") * 2 + lax.axis_index("c")
    pltpu.sync_copy(tbl_hbm, tbl_v)
    lanes = lax.iota(jnp.int32, 16)
    m0 = jnp.where(lanes == 0, 1.0, 0.0)
    m15 = jnp.where(lanes == 15, 1.0, 0.0)
    for bl in range(_BPW):
        b = wid * _BPW + bl
        pltpu.sync_copy(em_hbm.at[b], em_v)
        pltpu.sync_copy(tags_hbm.at[b], tg_v)

        def step(i, acc):
            t0 = i * 16
            tg = tg_v[pl.ds(t0, 16)]
            acc = acc + plsc.load_gather(em_v, [(t0 + lanes) * _K + tg])
            tgp = plsc.load_gather(tg_v, [jnp.maximum(t0 + lanes - 1, 0)])
            acc = acc + plsc.load_gather(tbl_v, [16 + tgp * _K + tg])
            return acc

        acc = lax.fori_loop(0, _T // 16, step, jnp.zeros((16,), jnp.float32))
        head = tg_v[pl.ds(0, 16)]
        tail = tg_v[pl.ds(_T - 16, 16)]
        g_start = plsc.load_gather(tbl_v, [head])
        g_end = plsc.load_gather(tbl_v, [8 + tail])
        g_t00 = plsc.load_gather(tbl_v, [16 + head * _K + head])
        acc = acc + (g_start - g_t00) * m0 + g_end * m15
        out_v[...] = acc
        pltpu.sync_copy(out_v, out_hbm.at[b])


def _post_sc(em2d, tags, tbl):
    mesh = plsc.VectorSubcoreMesh(core_axis_name="c", subcore_axis_name="s")
    fn = pl.kernel(
        _post_sc_body,
        mesh=mesh,
        compiler_params=pltpu.CompilerParams(needs_layout_passes=False),
        out_type=jax.ShapeDtypeStruct((_B, 16), jnp.float32),
        scratch_types=[
            pltpu.VMEM((_T * _K,), jnp.float32),
            pltpu.VMEM((_T,), jnp.int32),
            pltpu.VMEM((64,), jnp.float32),
            pltpu.VMEM((16,), jnp.float32),
        ],
    )
    return fn(em2d, tags, tbl)


def _z_body(em_ref, start_ref, trans_ref, transt_ref, end_ref, out_ref):
    wt = jnp.exp(transt_ref[...])
    wn = jnp.exp(trans_ref[...])
    colsB = [jnp.broadcast_to(wt[:, j:j + 1], (_K, _B)) for j in range(_K)]
    rowsB = [jnp.broadcast_to(wn[:, k:k + 1], (_K, _B)) for k in range(_K)]
    startB = jnp.broadcast_to(start_ref[...], (_K, _B))
    endB = jnp.broadcast_to(end_ref[...], (_K, _B))

    def matvec_f(p):
        acc = None
        for j in range(_K):
            c = p[j:j + 1, :] * colsB[j]
            acc = c if acc is None else acc + c
        return acc

    def matvec_b(y):
        acc = None
        for k in range(_K):
            c = y[k:k + 1, :] * rowsB[k]
            acc = c if acc is None else acc + c
        return acc

    def rescale(p, m):
        s = jnp.max(p, axis=0, keepdims=True)
        return p * (1.0 / s), m + jnp.log(s)

    def fwd_chunk(p, mf, ech2d, first):
        E2d = jnp.exp(ech2d)
        for i in range(_C):
            Ei = E2d[_K * i:_K * (i + 1)]
            if first and i == 0:
                p = jnp.exp(startB) * Ei
            else:
                p = matvec_f(p) * Ei
        return rescale(p, mf)

    def bwd_chunk(u, mb, ech2d):
        E2d = jnp.exp(ech2d)
        for i in range(_C - 1, -1, -1):
            u = matvec_b(u * E2d[_K * i:_K * (i + 1)])
        return rescale(u, mb)

    p, mf = fwd_chunk(None, jnp.zeros((1, _B), jnp.float32),
                      em_ref[0:_R], True)
    tb0 = _T - _C
    u, mb = bwd_chunk(jnp.exp(endB), jnp.zeros((1, _B), jnp.float32),
                      em_ref[tb0 * _K:_T * _K])

    def body(c, carry):
        p, mf, u, mb = carry
        p, mf = fwd_chunk(p, mf, em_ref[pl.ds(c * _C * _K, _R)], False)
        tb = _T - _C * (c + 1)
        u, mb = bwd_chunk(u, mb, em_ref[pl.ds(tb * _K, _R)])
        return p, mf, u, mb

    p, mf, u, mb = jax.lax.fori_loop(1, _NHALF, body, (p, mf, u, mb))
    out_ref[...] = mf + mb + jnp.log(jnp.sum(p * u, axis=0, keepdims=True))


def _z_pallas(emT2d, startC, trans, transT, endC, *, interpret=False):
    return pl.pallas_call(
        _z_body,
        out_shape=jax.ShapeDtypeStruct((1, _B), jnp.float32),
        in_specs=[
            pl.BlockSpec(memory_space=pltpu.VMEM),
            pl.BlockSpec(memory_space=pltpu.VMEM),
            pl.BlockSpec(memory_space=pltpu.VMEM),
            pl.BlockSpec(memory_space=pltpu.VMEM),
            pl.BlockSpec(memory_space=pltpu.VMEM),
        ],
        out_specs=pl.BlockSpec(memory_space=pltpu.VMEM),
        interpret=interpret,
    )(emT2d, startC, trans, transT, endC)


def kernel(emissions, mask, tags, start_transitions, transitions,
           end_transitions):
    del mask
    emT2d = jnp.transpose(emissions.reshape(_B, _T * _K), (1, 0))
    startC = start_transitions.reshape(_K, 1)
    endC = end_transitions.reshape(_K, 1)
    transT = jnp.transpose(transitions, (1, 0))
    tbl = jnp.zeros((64,), jnp.float32)
    tbl = tbl.at[0:_K].set(start_transitions)
    tbl = tbl.at[8:8 + _K].set(end_transitions)
    tbl = tbl.at[16:16 + _K * _K].set(transitions.reshape(-1))

    post = _post_sc(emissions.reshape(_B, _T * _K), tags, tbl)
    z = _z_pallas(emT2d, startC, transitions, transT, endC)
    return (jnp.sum(post) - jnp.sum(z)) / _B
